# baseline (speedup 1.0000x reference)
import jax
import jax.numpy as jnp
from jax import lax
from jax.experimental import pallas as pl
from jax.experimental.pallas import tpu as pltpu
from jax.experimental.pallas import tpu_sc as plsc

_LANES = 16
_NW = 32
_WIN = 10
_BAND = 2 * _WIN + 1


def _dtw_knn_body(a_hbm, fit_hbm, lab_hbm, out_hbm, a_v, fit_v, lab_v, res_v):
    t = fit_hbm.shape[0] // lab_v.shape[0]
    nkeys = lab_v.shape[0]
    wid = lax.axis_index("s") * 2 + lax.axis_index("c")
    pltpu.sync_copy(a_hbm.at[wid], a_v)
    pltpu.sync_copy(fit_hbm, fit_v)
    pltpu.sync_copy(lab_hbm, lab_v)

    inf = jnp.full((_LANES,), jnp.inf, jnp.float32)
    nflat = fit_v.shape[0]

    def bcast_b(idx, off=0):
        aligned = (off // 8) * 8
        if aligned:
            ref = fit_v.at[pl.ds(aligned, nflat - aligned)]
        else:
            ref = fit_v
        rem = off - aligned
        return plsc.load_gather(ref, [idx + rem if rem else idx])

    _NI = 2

    def key_body(jp, carry):
        best, besti = carry
        kidx = [jnp.full((_LANES,), _NI * jp + q, jnp.int32) for q in range(_NI)]
        kbase = [ki * t for ki in kidx]
        klim = [kb + (t - 1) for kb in kbase]

        a0 = a_v[pl.ds(0, _LANES)]
        st = [[inf] * _BAND for _ in range(_NI)]
        for q in range(_NI):
            run = jnp.abs(a0 - bcast_b(kbase[q]))
            st[q][_WIN] = run
            for k in range(_WIN + 1, _BAND):
                run = run + jnp.abs(a0 - bcast_b(kbase[q], k - _WIN))
                st[q][k] = run

        def make_row(clamp_lo, clamp_hi):
            def row_body(i, carry_t):
                rb = list(carry_t[:_NI])
                st = [list(carry_t[_NI + q * _BAND:_NI + (q + 1) * _BAND])
                      for q in range(_NI)]
                ai = a_v[pl.ds(i * _LANES, _LANES)]
                for k in range(_BAND - 1):
                    for q in range(_NI):
                        if clamp_lo and k < _WIN:
                            idx = rb[q] + k if k else rb[q]
                            c = jnp.abs(ai - bcast_b(jnp.maximum(idx, kbase[q])))
                        elif clamp_hi and k > _WIN:
                            idx = rb[q] + k
                            c = jnp.abs(ai - bcast_b(jnp.minimum(idx, klim[q])))
                        else:
                            idx = None
                            c = jnp.abs(ai - bcast_b(rb[q], k))
                        left = st[q][k - 1] if k >= 1 else inf
                        val = jnp.minimum(
                            jnp.minimum(st[q][k], st[q][k + 1]), left) + c
                        if clamp_lo and k < _WIN:
                            val = jnp.where(idx >= kbase[q], val, inf)
                        elif clamp_hi and k > _WIN:
                            val = jnp.where(idx <= klim[q], val, inf)
                        st[q][k] = val
                for q in range(_NI):
                    st[q][_BAND - 1] = inf
                return tuple(r + 1 for r in rb) + tuple(
                    v for q in range(_NI) for v in st[q])
            return row_body

        carry_t = tuple(kb + (1 - _WIN) for kb in kbase) + tuple(
            v for q in range(_NI) for v in st[q])
        carry_t = lax.fori_loop(1, _WIN + 1, make_row(True, False), carry_t)
        carry_t = lax.fori_loop(_WIN + 1, t - _WIN + 1, make_row(False, False),
                                carry_t)
        carry_t = lax.fori_loop(t - _WIN + 1, t, make_row(False, True), carry_t)

        for q in range(_NI):
            dist = carry_t[_NI + q * _BAND + _WIN]
            upd = dist < best
            best = jnp.where(upd, dist, best)
            besti = jnp.where(upd, kidx[q], besti)
        return best, besti

    best, besti = lax.fori_loop(
        0, nkeys // _NI,
        key_body,
        (inf, jnp.zeros((_LANES,), jnp.int32)),
    )
    res_v[...] = plsc.load_gather(lab_v, [besti])
    pltpu.sync_copy(res_v, out_hbm.at[pl.ds(wid * _LANES, _LANES)])


def kernel(samples, train_data, train_labels):
    fit_data = train_data[::100]
    fit_labels = train_labels[::100]
    s, t = samples.shape
    per_w = s // _NW
    a_resh = samples.reshape(_NW, per_w, t).transpose(0, 2, 1).reshape(_NW, -1)
    fit_flat = fit_data.reshape(-1)
    mesh = plsc.VectorSubcoreMesh(core_axis_name="c", subcore_axis_name="s")
    f = pl.kernel(
        _dtw_knn_body,
        out_type=jax.ShapeDtypeStruct((s,), jnp.int32),
        mesh=mesh,
        compiler_params=pltpu.CompilerParams(needs_layout_passes=False),
        scratch_types=[
            pltpu.VMEM((t * per_w,), jnp.float32),
            pltpu.VMEM((fit_data.shape[0] * t,), jnp.float32),
            pltpu.VMEM(fit_labels.shape, jnp.int32),
            pltpu.VMEM((per_w,), jnp.int32),
        ],
    )
    return f(a_resh, fit_flat, fit_labels)
